# baseline (speedup 1.0000x reference)
import functools

import jax
import jax.numpy as jnp
from jax import lax
from jax.experimental import pallas as pl
from jax.experimental.pallas import tpu as pltpu
from jax.experimental.pallas import tpu_sc as plsc

_HALF = 32

_TROWS = _HALF * 128 * 56
_TCOLS = 56
_TCH = 8192
_TNCHUNKS = _TROWS // _TCH
_TNBUF = 7


def _tc_body(x_hbm, o_hbm, *scratch):
    bufs = scratch[:_TNBUF]
    in_sems = scratch[_TNBUF:2 * _TNBUF]
    out_sems = scratch[2 * _TNBUF:]
    xf = x_hbm.reshape(64 * 128 * 56, _TCOLS)
    of = o_hbm.reshape(_TROWS, _TCOLS)

    def in_copy(i):
        s = i % _TNBUF
        return pltpu.make_async_copy(
            xf.at[pl.ds(i * _TCH, _TCH), :], bufs[s], in_sems[s]
        )

    def out_copy(i):
        s = i % _TNBUF
        return pltpu.make_async_copy(
            bufs[s], of.at[pl.ds(i * _TCH, _TCH), :], out_sems[s]
        )

    for i in range(min(_TNBUF, _TNCHUNKS)):
        in_copy(i).start()
    for i in range(_TNCHUNKS):
        in_copy(i).wait()
        out_copy(i).start()
        nxt = i + _TNBUF
        if nxt < _TNCHUNKS:
            out_copy(i).wait()
            in_copy(nxt).start()
    for i in range(max(0, _TNCHUNKS - _TNBUF), _TNCHUNKS):
        out_copy(i).wait()


def _tc_half(x):
    return pl.pallas_call(
        _tc_body,
        in_specs=[pl.BlockSpec(memory_space=pl.ANY)],
        out_specs=pl.BlockSpec(memory_space=pl.ANY),
        out_shape=jax.ShapeDtypeStruct((_HALF, 128, 56, 56), x.dtype),
        scratch_shapes=(
            [pltpu.VMEM((_TCH, _TCOLS), jnp.float32) for _ in range(_TNBUF)]
            + [pltpu.SemaphoreType.DMA(()) for _ in range(2 * _TNBUF)]
        ),
    )(x)


_NC = 2
_NS = 16
_NW = _NC * _NS
_SNBUF = 4
_SNCHUNK = 32
_SC1 = 128 // _SNCHUNK


def _sc_body(x_hbm, o_hbm, *scratch):
    bufs = scratch[:_SNBUF]
    in_sems = scratch[_SNBUF:2 * _SNBUF]
    out_sems = scratch[2 * _SNBUF:]
    wid = lax.axis_index("s") * _NC + lax.axis_index("c")
    src_row = _HALF + wid

    def in_copy(j):
        b = j % _SNBUF
        return pltpu.make_async_copy(
            x_hbm.at[pl.ds(src_row, 1), pl.ds(j * _SC1, _SC1)], bufs[b], in_sems[b]
        )

    def out_copy(j):
        b = j % _SNBUF
        return pltpu.make_async_copy(
            bufs[b], o_hbm.at[pl.ds(wid, 1), pl.ds(j * _SC1, _SC1)], out_sems[b]
        )

    for j in range(min(_SNBUF, _SNCHUNK)):
        in_copy(j).start()
    for j in range(_SNCHUNK):
        in_copy(j).wait()
        out_copy(j).start()
        nxt = j + _SNBUF
        if nxt < _SNCHUNK:
            out_copy(j).wait()
            in_copy(nxt).start()
    for j in range(max(0, _SNCHUNK - _SNBUF), _SNCHUNK):
        out_copy(j).wait()


def _sc_half(x):
    run = functools.partial(
        pl.kernel,
        mesh=plsc.VectorSubcoreMesh(core_axis_name="c", subcore_axis_name="s"),
        out_type=jax.ShapeDtypeStruct((_HALF, 128, 56, 56), x.dtype),
        scratch_types=(
            [pltpu.VMEM((1, _SC1, 56, 56), jnp.float32) for _ in range(_SNBUF)]
            + [pltpu.SemaphoreType.DMA for _ in range(2 * _SNBUF)]
        ),
    )(_sc_body)
    return run(x)


def kernel(x):
    a = _tc_half(x)
    b = _sc_half(x)
    return jnp.concatenate([a, b], axis=0)


# speedup vs baseline: 1.0013x; 1.0013x over previous
import functools

import jax
import jax.numpy as jnp
from jax import lax
from jax.experimental import pallas as pl
from jax.experimental.pallas import tpu as pltpu
from jax.experimental.pallas import tpu_sc as plsc

_HALF = 32

_TROWS = _HALF * 128 * 56
_TCOLS = 56
_TCH = 8192
_TNCHUNKS = _TROWS // _TCH
_TNBUF = 7


def _tc_body(x_hbm, o_hbm, *scratch):
    bufs = scratch[:_TNBUF]
    in_sems = scratch[_TNBUF:2 * _TNBUF]
    out_sems = scratch[2 * _TNBUF:]
    xf = x_hbm.reshape(64 * 128 * 56, _TCOLS)
    of = o_hbm.reshape(_TROWS, _TCOLS)

    def in_copy(i):
        s = i % _TNBUF
        return pltpu.make_async_copy(
            xf.at[pl.ds(i * _TCH, _TCH), :], bufs[s], in_sems[s]
        )

    def out_copy(i):
        s = i % _TNBUF
        return pltpu.make_async_copy(
            bufs[s], of.at[pl.ds(i * _TCH, _TCH), :], out_sems[s]
        )

    for i in range(min(_TNBUF, _TNCHUNKS)):
        in_copy(i).start()
    for i in range(_TNCHUNKS):
        in_copy(i).wait()
        out_copy(i).start()
        nxt = i + _TNBUF
        if nxt < _TNCHUNKS:
            out_copy(i).wait()
            in_copy(nxt).start()
    for i in range(max(0, _TNCHUNKS - _TNBUF), _TNCHUNKS):
        out_copy(i).wait()


def _tc_half(x):
    return pl.pallas_call(
        _tc_body,
        in_specs=[pl.BlockSpec(memory_space=pl.ANY)],
        out_specs=pl.BlockSpec(memory_space=pl.ANY),
        out_shape=jax.ShapeDtypeStruct((_HALF, 128, 56, 56), x.dtype),
        scratch_shapes=(
            [pltpu.VMEM((_TCH, _TCOLS), jnp.float32) for _ in range(_TNBUF)]
            + [pltpu.SemaphoreType.DMA(()) for _ in range(2 * _TNBUF)]
        ),
    )(x)


_NC = 2
_NS = 16
_NW = _NC * _NS
_SNBUF = 4
_SNCHUNK = 32
_SC1 = 128 // _SNCHUNK


def _sc_body(x_hbm, o_hbm, *scratch):
    bufs = scratch[:_SNBUF]
    in_sems = scratch[_SNBUF:2 * _SNBUF]
    out_sems = scratch[2 * _SNBUF:]
    wid = lax.axis_index("s") * _NC + lax.axis_index("c")
    src_row = _HALF + wid

    def in_copy(j):
        b = j % _SNBUF
        return pltpu.make_async_copy(
            x_hbm.at[pl.ds(src_row, 1), pl.ds(j * _SC1, _SC1)], bufs[b], in_sems[b]
        )

    def out_copy(j):
        b = j % _SNBUF
        return pltpu.make_async_copy(
            bufs[b], o_hbm.at[pl.ds(wid, 1), pl.ds(j * _SC1, _SC1)], out_sems[b]
        )

    for j in range(min(_SNBUF, _SNCHUNK)):
        in_copy(j).start()
    for j in range(_SNCHUNK):
        in_copy(j).wait()
        out_copy(j).start()
        nxt = j + _SNBUF
        if nxt < _SNCHUNK:
            out_copy(j).wait()
            in_copy(nxt).start()
    for j in range(max(0, _SNCHUNK - _SNBUF), _SNCHUNK):
        out_copy(j).wait()


def _sc_half(x):
    run = functools.partial(
        pl.kernel,
        mesh=plsc.VectorSubcoreMesh(core_axis_name="c", subcore_axis_name="s"),
        out_type=jax.ShapeDtypeStruct((_HALF, 128, 56, 56), x.dtype),
        scratch_types=(
            [pltpu.VMEM((1, _SC1, 56, 56), jnp.float32) for _ in range(_SNBUF)]
            + [pltpu.SemaphoreType.DMA for _ in range(2 * _SNBUF)]
        ),
    )(_sc_body)
    return run(x)


def kernel(x):
    b = _sc_half(x)
    a = _tc_half(x)
    return jnp.concatenate([a, b], axis=0)
